# baseline (speedup 1.0000x reference)
import functools

import jax
import jax.numpy as jnp
from jax import lax
from jax.experimental import pallas as pl
from jax.experimental.pallas import tpu as pltpu
from jax.experimental.pallas import tpu_sc as plsc

_NC, _NS = 2, 16
_NW = _NC * _NS
_EPS = 1e-12
_C = 4
_BB = 64


def _pick_gw(per_w):
    for gw in range(128, 0, -8):
        if per_w % gw == 0:
            return gw
    return 8


def _sc_gather_chunk(table64, nrows, hw, flat_ids, tok_off, ntok):
    per_w = ntok // _NW
    gw = _pick_gw(per_w)
    nch = per_w // gw
    mesh = plsc.VectorSubcoreMesh(core_axis_name="c", subcore_axis_name="s")

    @functools.partial(
        pl.kernel,
        out_type=jax.ShapeDtypeStruct((ntok, hw), jnp.uint32),
        mesh=mesh,
        scratch_types=[
            pltpu.VMEM((per_w,), jnp.int32),
            pltpu.VMEM((gw, hw), jnp.uint32),
            pltpu.VMEM((gw, hw), jnp.uint32),
            pltpu.SemaphoreType.DMA,
            pltpu.SemaphoreType.DMA,
        ],
        compiler_params=pltpu.CompilerParams(use_tc_tiling_on_sc=False),
    )
    def gather_kernel(table_hbm, idx_hbm, out_hbm, idx_v, buf0, buf1,
                      gsem0, gsem1):
        wid = lax.axis_index("s") * _NC + lax.axis_index("c")
        base = wid * per_w
        pltpu.sync_copy(idx_hbm.at[pl.ds(tok_off + base, per_w)], idx_v)

        def start(c, buf, sem):
            return pltpu.async_copy(
                table_hbm.at[idx_v.at[pl.ds(c * gw, gw)]], buf, sem)

        def wait(c, buf, sem):
            pltpu.make_async_copy(
                table_hbm.at[idx_v.at[pl.ds(c * gw, gw)]], buf, sem).wait()

        def writeout(c, buf):
            pltpu.sync_copy(buf, out_hbm.at[pl.ds(base + c * gw, gw)])

        start(0, buf0, gsem0)

        @pl.loop(0, nch, step=2)
        def _(c):
            @pl.when(c + 1 < nch)
            def _():
                start(c + 1, buf1, gsem1)

            wait(c, buf0, gsem0)
            writeout(c, buf0)

            @pl.when(c + 2 < nch)
            def _():
                start(c + 2, buf0, gsem0)

            @pl.when(c + 1 < nch)
            def _():
                wait(c + 1, buf1, gsem1)
                writeout(c + 1, buf1)

    return gather_kernel(table64, flat_ids)


def _tc_norm_chunk(big, staged2, token_type_ids, pos3, type_emb,
                   gm2, bt2, chunk, nchunks):
    b, l = token_type_ids.shape
    h = staged2.shape[1]
    bc = b // nchunks
    blk0 = chunk * (bc // _BB)
    rows2 = _BB * l // 2

    def body(*refs):
        g_ref, tt_ref, pos_ref, te_ref, gm_ref, bt_ref, o_ref = refs[-7:]
        xi = g_ref[...]
        lo = lax.bitcast_convert_type(xi << jnp.uint32(16), jnp.float32)
        hi = lax.bitcast_convert_type(xi & jnp.uint32(0xFFFF0000),
                                      jnp.float32)
        lo3 = lo.reshape(rows2, 2, h // 2)
        hi3 = hi.reshape(rows2, 2, h // 2)
        x = jnp.concatenate([lo3, hi3], axis=-1).reshape(_BB, l, h)
        x = x + pos_ref[...]
        tt = tt_ref[...].astype(jnp.float32)[..., None]
        te0 = te_ref[0:1, :].reshape(1, 1, h)
        te1 = te_ref[1:2, :].reshape(1, 1, h)
        x = x + te0 + tt * (te1 - te0)
        x2 = x.reshape(_BB * l, h)
        xb = x2.astype(jnp.bfloat16)
        ones = jnp.ones((h, h), jnp.bfloat16)
        mm = lambda a: lax.dot_general(
            a, ones, (((1,), (0,)), ((), ())),
            preferred_element_type=jnp.float32)
        mean = mm(xb) * (1.0 / h)
        ex2 = mm(xb * xb) * (1.0 / h)
        var = ex2 - mean * mean
        y = (x2 - mean) * lax.rsqrt(var + _EPS)
        y = y * gm_ref[...].reshape(1, h) + bt_ref[...].reshape(1, h)
        o_ref[...] = y.reshape(_BB, l, h)

    in_specs = [
        pl.BlockSpec((rows2, h), lambda i: (i, 0)),
        pl.BlockSpec((_BB, l), lambda i, _b0=blk0: (i + _b0, 0)),
        pl.BlockSpec((1, l, h), lambda i: (0, 0, 0)),
        pl.BlockSpec((2, h), lambda i: (0, 0)),
        pl.BlockSpec((1, h), lambda i: (0, 0)),
        pl.BlockSpec((1, h), lambda i: (0, 0)),
    ]
    args = [staged2, token_type_ids, pos3, type_emb, gm2, bt2]
    aliases = {}
    if big is not None:
        in_specs = [pl.BlockSpec(memory_space=pl.ANY)] + in_specs
        args = [big] + args
        aliases = {0: 0}

    return pl.pallas_call(
        body,
        grid=(bc // _BB,),
        in_specs=in_specs,
        out_specs=pl.BlockSpec((_BB, l, h),
                               lambda i, _b0=blk0: (i + _b0, 0, 0)),
        out_shape=jax.ShapeDtypeStruct((b, l, h), jnp.float32),
        input_output_aliases=aliases,
        compiler_params=pltpu.CompilerParams(
            dimension_semantics=("arbitrary",)),
    )(*args)


def kernel(input_ids, token_type_ids, word_emb, pos_emb, type_emb, gamma, beta):
    b, l = input_ids.shape
    v, h = word_emb.shape
    hw = h // 2
    flat_ids = input_ids.reshape(b * l)
    pos3 = pos_emb[:l].reshape(1, l, h)
    gm2 = gamma.reshape(1, h)
    bt2 = beta.reshape(1, h)
    wb = word_emb.astype(jnp.bfloat16)
    lo16 = lax.bitcast_convert_type(wb[:, :hw], jnp.uint16)
    hi16 = lax.bitcast_convert_type(wb[:, hw:], jnp.uint16)
    table64 = lo16.astype(jnp.uint32) | (hi16.astype(jnp.uint32) << 16)
    bc = b // _C
    staged = [
        _sc_gather_chunk(table64, v, hw, flat_ids, c * bc * l, bc * l)
        .reshape(bc * l // 2, h)
        for c in range(_C)
    ]
    out = None
    for c in range(_C):
        out = _tc_norm_chunk(out, staged[c], token_type_ids, pos3,
                             type_emb, gm2, bt2, c, _C)
    return out


# speedup vs baseline: 3.0522x; 3.0522x over previous
import functools

import jax
import jax.numpy as jnp
from jax import lax
from jax.experimental import pallas as pl
from jax.experimental.pallas import tpu as pltpu
from jax.experimental.pallas import tpu_sc as plsc

_NC, _NS = 2, 16
_NW = _NC * _NS
_EPS = 1e-12
_C = 4
_BB = 64


def _pick_gw(per_w):
    for gw in range(128, 0, -8):
        if per_w % gw == 0:
            return gw
    return 8


def _sc_gather_chunk(word_emb, flat_ids, chunk, nchunks):
    n = flat_ids.shape[0]
    h = word_emb.shape[1]
    nc = n // nchunks
    per_w = nc // _NW
    gw = _pick_gw(per_w)
    nch = per_w // gw
    chunk_off = chunk * nc
    mesh = plsc.VectorSubcoreMesh(core_axis_name="c", subcore_axis_name="s")

    @functools.partial(
        pl.kernel,
        out_type=jax.ShapeDtypeStruct((nc, h), jnp.float32),
        mesh=mesh,
        scratch_types=[
            pltpu.VMEM((per_w,), jnp.int32),
            pltpu.VMEM((gw, h), jnp.float32),
            pltpu.VMEM((gw, h), jnp.float32),
            pltpu.SemaphoreType.DMA,
            pltpu.SemaphoreType.DMA,
        ],
    )
    def gather_kernel(table_hbm, idx_hbm, out_hbm, idx_v, buf0, buf1,
                      gsem0, gsem1):
        wid = lax.axis_index("s") * _NC + lax.axis_index("c")
        base = wid * per_w
        pltpu.sync_copy(idx_hbm.at[pl.ds(chunk_off + base, per_w)], idx_v)

        def start(c, buf, sem):
            return pltpu.async_copy(
                table_hbm.at[idx_v.at[pl.ds(c * gw, gw)]], buf, sem)

        def wait(c, buf, sem):
            pltpu.make_async_copy(
                table_hbm.at[idx_v.at[pl.ds(c * gw, gw)]], buf, sem).wait()

        def writeout(c, buf):
            pltpu.sync_copy(buf, out_hbm.at[pl.ds(base + c * gw, gw)])

        start(0, buf0, gsem0)

        @pl.loop(0, nch, step=2)
        def _(c):
            @pl.when(c + 1 < nch)
            def _():
                start(c + 1, buf1, gsem1)

            wait(c, buf0, gsem0)
            writeout(c, buf0)

            @pl.when(c + 2 < nch)
            def _():
                start(c + 2, buf0, gsem0)

            @pl.when(c + 1 < nch)
            def _():
                wait(c + 1, buf1, gsem1)
                writeout(c + 1, buf1)

    return gather_kernel(word_emb, flat_ids)


def _tc_norm_chunk(big, gathered3c, token_type_ids, pos3, type_emb,
                   gm2, bt2, chunk, nchunks):
    b, l = token_type_ids.shape
    h = gathered3c.shape[2]
    bc = b // nchunks
    blk0 = chunk * (bc // _BB)

    def body(*refs):
        g_ref, tt_ref, pos_ref, te_ref, gm_ref, bt_ref, o_ref = refs[-7:]
        x = g_ref[...] + pos_ref[...]
        tt = tt_ref[...].astype(jnp.float32)[..., None]
        te0 = te_ref[0:1, :].reshape(1, 1, h)
        te1 = te_ref[1:2, :].reshape(1, 1, h)
        x = x + te0 + tt * (te1 - te0)
        x2 = x.reshape(_BB * l, h)
        xb = x2.astype(jnp.bfloat16)
        ones = jnp.ones((h, h), jnp.bfloat16)
        mm = lambda a: lax.dot_general(
            a, ones, (((1,), (0,)), ((), ())),
            preferred_element_type=jnp.float32)
        mean = mm(xb) * (1.0 / h)
        ex2 = mm(xb * xb) * (1.0 / h)
        var = ex2 - mean * mean
        y = (x2 - mean) * lax.rsqrt(var + _EPS)
        y = y * gm_ref[...].reshape(1, h) + bt_ref[...].reshape(1, h)
        o_ref[...] = y.reshape(_BB, l, h)

    in_specs = [
        pl.BlockSpec((_BB, l, h), lambda i: (i, 0, 0)),
        pl.BlockSpec((_BB, l), lambda i, _b0=blk0: (i + _b0, 0)),
        pl.BlockSpec((1, l, h), lambda i: (0, 0, 0)),
        pl.BlockSpec((2, h), lambda i: (0, 0)),
        pl.BlockSpec((1, h), lambda i: (0, 0)),
        pl.BlockSpec((1, h), lambda i: (0, 0)),
    ]
    args = [gathered3c, token_type_ids, pos3, type_emb, gm2, bt2]
    aliases = {}
    if big is not None:
        in_specs = [pl.BlockSpec(memory_space=pl.ANY)] + in_specs
        args = [big] + args
        aliases = {0: 0}

    return pl.pallas_call(
        body,
        grid=(bc // _BB,),
        in_specs=in_specs,
        out_specs=pl.BlockSpec((_BB, l, h),
                               lambda i, _b0=blk0: (i + _b0, 0, 0)),
        out_shape=jax.ShapeDtypeStruct((b, l, h), jnp.float32),
        input_output_aliases=aliases,
        compiler_params=pltpu.CompilerParams(
            dimension_semantics=("arbitrary",)),
    )(*args)


def kernel(input_ids, token_type_ids, word_emb, pos_emb, type_emb, gamma, beta):
    b, l = input_ids.shape
    h = word_emb.shape[1]
    flat_ids = input_ids.reshape(b * l)
    pos3 = pos_emb[:l].reshape(1, l, h)
    gm2 = gamma.reshape(1, h)
    bt2 = beta.reshape(1, h)
    gathered = [
        _sc_gather_chunk(word_emb, flat_ids, c, _C).reshape(b // _C, l, h)
        for c in range(_C)
    ]
    out = None
    for c in range(_C):
        out = _tc_norm_chunk(out, gathered[c], token_type_ids, pos3,
                             type_emb, gm2, bt2, c, _C)
    return out


# speedup vs baseline: 3.1522x; 1.0327x over previous
import functools

import jax
import jax.numpy as jnp
from jax import lax
from jax.experimental import pallas as pl
from jax.experimental.pallas import tpu as pltpu
from jax.experimental.pallas import tpu_sc as plsc

_NC, _NS = 2, 16
_NW = _NC * _NS
_EPS = 1e-12
_C = 4
_BB = 64


def _pick_gw(per_w):
    for gw in range(128, 0, -8):
        if per_w % gw == 0:
            return gw
    return 8


def _sc_gather_chunk(word_emb, flat_ids, chunk, nchunks):
    n = flat_ids.shape[0]
    h = word_emb.shape[1]
    nc = n // nchunks
    per_w = nc // _NW
    gw = _pick_gw(per_w)
    nch = per_w // gw
    chunk_off = chunk * nc
    mesh = plsc.VectorSubcoreMesh(core_axis_name="c", subcore_axis_name="s")

    @functools.partial(
        pl.kernel,
        out_type=jax.ShapeDtypeStruct((nc, h), jnp.float32),
        mesh=mesh,
        scratch_types=[
            pltpu.VMEM((per_w,), jnp.int32),
            pltpu.VMEM((gw, h), jnp.float32),
            pltpu.VMEM((gw, h), jnp.float32),
            pltpu.VMEM((gw, h), jnp.float32),
            pltpu.VMEM((gw, h), jnp.float32),
            pltpu.SemaphoreType.DMA,
            pltpu.SemaphoreType.DMA,
            pltpu.SemaphoreType.DMA,
            pltpu.SemaphoreType.DMA,
        ],
    )
    def gather_kernel(table_hbm, idx_hbm, out_hbm, idx_v,
                      buf0, buf1, buf2, buf3, sem0, sem1, sem2, sem3):
        wid = lax.axis_index("s") * _NC + lax.axis_index("c")
        base = wid * per_w
        bufs = (buf0, buf1, buf2, buf3)
        sems = (sem0, sem1, sem2, sem3)
        pltpu.sync_copy(idx_hbm.at[pl.ds(chunk_off + base, per_w)], idx_v)

        def start(c, j):
            return pltpu.async_copy(
                table_hbm.at[idx_v.at[pl.ds(c * gw, gw)]], bufs[j], sems[j])

        def wait(c, j):
            pltpu.make_async_copy(
                table_hbm.at[idx_v.at[pl.ds(c * gw, gw)]],
                bufs[j], sems[j]).wait()

        def writeout(c, j):
            pltpu.sync_copy(bufs[j], out_hbm.at[pl.ds(base + c * gw, gw)])

        for j in range(min(4, nch)):
            start(j, j)

        @pl.loop(0, nch, step=4)
        def _(c):
            for j in range(4):
                @pl.when(c + j < nch)
                def _(j=j):
                    wait(c + j, j)
                    writeout(c + j, j)

                    @pl.when(c + j + 4 < nch)
                    def _(j=j):
                        start(c + j + 4, j)

    return gather_kernel(word_emb, flat_ids)


def _tc_norm_chunk(big, gathered3c, token_type_ids, pos3, type_emb,
                   gm2, bt2, chunk, nchunks):
    b, l = token_type_ids.shape
    h = gathered3c.shape[2]
    bc = b // nchunks
    blk0 = chunk * (bc // _BB)

    def body(*refs):
        g_ref, tt_ref, pos_ref, te_ref, gm_ref, bt_ref, o_ref = refs[-7:]
        x = g_ref[...] + pos_ref[...]
        tt = tt_ref[...].astype(jnp.float32)[..., None]
        te0 = te_ref[0:1, :].reshape(1, 1, h)
        te1 = te_ref[1:2, :].reshape(1, 1, h)
        x = x + te0 + tt * (te1 - te0)
        x2 = x.reshape(_BB * l, h)
        xb = x2.astype(jnp.bfloat16)
        ones = jnp.ones((h, h), jnp.bfloat16)
        mm = lambda a: lax.dot_general(
            a, ones, (((1,), (0,)), ((), ())),
            preferred_element_type=jnp.float32)
        mean = mm(xb) * (1.0 / h)
        ex2 = mm(xb * xb) * (1.0 / h)
        var = ex2 - mean * mean
        y = (x2 - mean) * lax.rsqrt(var + _EPS)
        y = y * gm_ref[...].reshape(1, h) + bt_ref[...].reshape(1, h)
        o_ref[...] = y.reshape(_BB, l, h)

    in_specs = [
        pl.BlockSpec((_BB, l, h), lambda i: (i, 0, 0)),
        pl.BlockSpec((_BB, l), lambda i, _b0=blk0: (i + _b0, 0)),
        pl.BlockSpec((1, l, h), lambda i: (0, 0, 0)),
        pl.BlockSpec((2, h), lambda i: (0, 0)),
        pl.BlockSpec((1, h), lambda i: (0, 0)),
        pl.BlockSpec((1, h), lambda i: (0, 0)),
    ]
    args = [gathered3c, token_type_ids, pos3, type_emb, gm2, bt2]
    aliases = {}
    if big is not None:
        in_specs = [pl.BlockSpec(memory_space=pl.ANY)] + in_specs
        args = [big] + args
        aliases = {0: 0}

    return pl.pallas_call(
        body,
        grid=(bc // _BB,),
        in_specs=in_specs,
        out_specs=pl.BlockSpec((_BB, l, h),
                               lambda i, _b0=blk0: (i + _b0, 0, 0)),
        out_shape=jax.ShapeDtypeStruct((b, l, h), jnp.float32),
        input_output_aliases=aliases,
        compiler_params=pltpu.CompilerParams(
            dimension_semantics=("arbitrary",)),
    )(*args)


def kernel(input_ids, token_type_ids, word_emb, pos_emb, type_emb, gamma, beta):
    b, l = input_ids.shape
    h = word_emb.shape[1]
    flat_ids = input_ids.reshape(b * l)
    pos3 = pos_emb[:l].reshape(1, l, h)
    gm2 = gamma.reshape(1, h)
    bt2 = beta.reshape(1, h)
    gathered = [
        _sc_gather_chunk(word_emb, flat_ids, c, _C).reshape(b // _C, l, h)
        for c in range(_C)
    ]
    out = None
    for c in range(_C):
        out = _tc_norm_chunk(out, gathered[c], token_type_ids, pos3,
                             type_emb, gm2, bt2, c, _C)
    return out
